# baseline (speedup 1.0000x reference)
import functools

import jax
import jax.numpy as jnp
from jax import lax
from jax.experimental import pallas as pl
from jax.experimental.pallas import tpu as pltpu
from jax.experimental.pallas import tpu_sc as plsc

N = 10000
E = 320000
D = 128
NS = 16
NW = 2 * NS
CH = 128
NCHUNK = 80
EPAD = NW * NCHUNK * CH
NP = 10240
ROWS_PT = NP // NS

_BLK = 2000


def _mm_body(h_ref, wl_ref, wr_ref, bl_ref, y_ref, z_ref):
    h = h_ref[...]
    dn = (((1,), (1,)), ((), ()))
    y_ref[...] = lax.dot_general(h, wl_ref[...], dn,
                                 preferred_element_type=jnp.float32)
    z_ref[...] = lax.dot_general(h, wr_ref[...], dn,
                                 preferred_element_type=jnp.float32) + bl_ref[...]


def _mm(h, wl, wr, bl):
    grid = (N // _BLK,)
    return pl.pallas_call(
        _mm_body,
        grid=grid,
        in_specs=[
            pl.BlockSpec((_BLK, D), lambda i: (i, 0)),
            pl.BlockSpec((D, D), lambda i: (0, 0)),
            pl.BlockSpec((D, D), lambda i: (0, 0)),
            pl.BlockSpec((1, D), lambda i: (0, 0)),
        ],
        out_specs=[
            pl.BlockSpec((_BLK, D), lambda i: (i, 0)),
            pl.BlockSpec((_BLK, D), lambda i: (i, 0)),
        ],
        out_shape=[
            jax.ShapeDtypeStruct((N, D), jnp.float32),
            jax.ShapeDtypeStruct((N, D), jnp.float32),
        ],
    )(h, wl, wr, bl.reshape(1, D))


def _combine_body(act, aggA_ref, aggB_ref, cntA_ref, cntB_ref, z_ref, o_ref):
    cnt = jnp.maximum(cntA_ref[...] + cntB_ref[...], 1.0)
    agg = aggA_ref[0] + aggB_ref[0]
    h = agg / cnt + z_ref[...]
    if act == "relu":
        h = jnp.maximum(h, 0.0)
    elif act == "logsoftmax":
        m = jnp.max(h, axis=1, keepdims=True)
        h = h - m
        h = h - jnp.log(jnp.sum(jnp.exp(h), axis=1, keepdims=True))
    o_ref[...] = h


def _combine(agg2, cntA, cntB, z, act):
    grid = (N // _BLK,)
    return pl.pallas_call(
        functools.partial(_combine_body, act),
        grid=grid,
        in_specs=[
            pl.BlockSpec((1, _BLK, D), lambda i: (0, i, 0)),
            pl.BlockSpec((1, _BLK, D), lambda i: (1, i, 0)),
            pl.BlockSpec((_BLK, 1), lambda i: (i, 0)),
            pl.BlockSpec((_BLK, 1), lambda i: (i, 0)),
            pl.BlockSpec((_BLK, D), lambda i: (i, 0)),
        ],
        out_specs=pl.BlockSpec((_BLK, D), lambda i: (i, 0)),
        out_shape=jax.ShapeDtypeStruct((N, D), jnp.float32),
    )(agg2, agg2, cntA, cntB, z)


def _sc_body(with_counts, *refs):
    if with_counts:
        (ys_h, src_h, dst_h, zrows_h, zcnt_h,
         agg_h, cntA_h, cntB_h,
         acc_s, cntacc_s,
         src_v, dst_v, rows0_v, ones_v) = refs
    else:
        (ys_h, src_h, dst_h, zrows_h,
         agg_h,
         acc_s,
         src_v, dst_v, rows0_v) = refs

    cid = lax.axis_index("c")
    sid = lax.axis_index("s")
    wid = cid * NS + sid

    pltpu.sync_copy(zrows_h, acc_s.at[pl.ds(sid * ROWS_PT, ROWS_PT)])
    if with_counts:
        pltpu.sync_copy(zcnt_h, cntacc_s.at[pl.ds(sid * ROWS_PT, ROWS_PT)])

    pltpu.sync_copy(src_h.at[wid], src_v)
    pltpu.sync_copy(dst_h.at[wid], dst_v)

    if with_counts:
        for k in range(CH // 16):
            ones_v[pl.ds(k * 16, 16)] = jnp.full((16,), 1.0, jnp.float32)

    plsc.subcore_barrier()

    def chunk(j, _):
        pltpu.sync_copy(ys_h.at[src_v.at[j]], rows0_v)
        pltpu.sync_copy(rows0_v, acc_s.at[dst_v.at[j]], add=True)
        if with_counts:
            pltpu.sync_copy(ones_v, cntacc_s.at[dst_v.at[j]], add=True)
        return 0

    lax.fori_loop(0, NCHUNK, chunk, 0)

    plsc.subcore_barrier()

    pltpu.sync_copy(acc_s.at[pl.ds(sid * ROWS_PT, ROWS_PT)],
                    agg_h.at[cid].at[pl.ds(sid * ROWS_PT, ROWS_PT)])

    if with_counts:
        @pl.when(cid == 0)
        def _():
            pltpu.sync_copy(cntacc_s.at[pl.ds(sid * ROWS_PT, ROWS_PT)],
                            cntA_h.at[pl.ds(sid * ROWS_PT, ROWS_PT)])

        @pl.when(cid == 1)
        def _():
            pltpu.sync_copy(cntacc_s.at[pl.ds(sid * ROWS_PT, ROWS_PT)],
                            cntB_h.at[pl.ds(sid * ROWS_PT, ROWS_PT)])


def _sc_agg(ys, src2, dst2, zrows, zcnt):
    mesh = plsc.VectorSubcoreMesh(core_axis_name="c", subcore_axis_name="s")
    f = pl.kernel(
        functools.partial(_sc_body, True),
        out_type=[
            jax.ShapeDtypeStruct((2, NP, D), jnp.float32),
            jax.ShapeDtypeStruct((NP,), jnp.float32),
            jax.ShapeDtypeStruct((NP,), jnp.float32),
        ],
        mesh=mesh,
        scratch_types=[
            pltpu.VMEM_SHARED((NP, D), jnp.float32),
            pltpu.VMEM_SHARED((NP,), jnp.float32),
            pltpu.VMEM((NCHUNK, CH), jnp.int32),
            pltpu.VMEM((NCHUNK, CH), jnp.int32),
            pltpu.VMEM((CH, D), jnp.float32),
            pltpu.VMEM((CH,), jnp.float32),
        ],
    )
    return f(ys, src2, dst2, zrows, zcnt)


def _sc_agg_nocnt(ys, src2, dst2, zrows):
    mesh = plsc.VectorSubcoreMesh(core_axis_name="c", subcore_axis_name="s")
    f = pl.kernel(
        functools.partial(_sc_body, False),
        out_type=jax.ShapeDtypeStruct((2, NP, D), jnp.float32),
        mesh=mesh,
        scratch_types=[
            pltpu.VMEM_SHARED((NP, D), jnp.float32),
            pltpu.VMEM((NCHUNK, CH), jnp.int32),
            pltpu.VMEM((NCHUNK, CH), jnp.int32),
            pltpu.VMEM((CH, D), jnp.float32),
        ],
    )
    return f(ys, src2, dst2, zrows)


def _layer(h, wl, wr, bl, act, src2, dst2, zrows, zcnt, cnts):
    y, z = _mm(h, wl, wr, bl)
    if cnts is None:
        agg2, cntA, cntB = _sc_agg(y, src2, dst2, zrows, zcnt)
        cnts = (cntA.reshape(NP, 1), cntB.reshape(NP, 1))
    else:
        agg2 = _sc_agg_nocnt(y, src2, dst2, zrows)
    return _combine(agg2, cnts[0], cnts[1], z, act), cnts


def kernel(x, edge_index, W1l, W1r, W2l, W2r, W3l, W3r, b1l, b2l, b3l):
    npad = EPAD - E
    srcp = jnp.concatenate([edge_index[0], jnp.zeros((npad,), edge_index.dtype)])
    dstp = jnp.concatenate([edge_index[1], jnp.full((npad,), N, edge_index.dtype)])
    src2 = srcp.reshape(NW, NCHUNK, CH)
    dst2 = dstp.reshape(NW, NCHUNK, CH)
    zrows = jnp.zeros((ROWS_PT, D), jnp.float32)
    zcnt = jnp.zeros((ROWS_PT,), jnp.float32)

    h, cnts = _layer(x, W1l, W1r, b1l, "relu", src2, dst2, zrows, zcnt, None)
    h, _ = _layer(h, W2l, W2r, b2l, "relu", src2, dst2, zrows, zcnt, cnts)
    out, _ = _layer(h, W3l, W3r, b3l, "logsoftmax", src2, dst2, zrows, zcnt, cnts)
    return out


# speedup vs baseline: 1.0018x; 1.0018x over previous
import functools

import jax
import jax.numpy as jnp
from jax import lax
from jax.experimental import pallas as pl
from jax.experimental.pallas import tpu as pltpu
from jax.experimental.pallas import tpu_sc as plsc

N = 10000
E = 320000
D = 128
NS = 16
NW = 2 * NS
CH = 128
NCHUNK = 80
EPAD = NW * NCHUNK * CH
NP = 10240
ROWS_PT = NP // NS

_BLK = 2000


def _mm_body(h_ref, wl_ref, wr_ref, bl_ref, y_ref, z_ref):
    h = h_ref[...]
    dn = (((1,), (1,)), ((), ()))
    y_ref[...] = lax.dot_general(h, wl_ref[...], dn,
                                 preferred_element_type=jnp.float32)
    z_ref[...] = lax.dot_general(h, wr_ref[...], dn,
                                 preferred_element_type=jnp.float32) + bl_ref[...]


def _mm(h, wl, wr, bl):
    grid = (N // _BLK,)
    return pl.pallas_call(
        _mm_body,
        grid=grid,
        in_specs=[
            pl.BlockSpec((_BLK, D), lambda i: (i, 0)),
            pl.BlockSpec((D, D), lambda i: (0, 0)),
            pl.BlockSpec((D, D), lambda i: (0, 0)),
            pl.BlockSpec((1, D), lambda i: (0, 0)),
        ],
        out_specs=[
            pl.BlockSpec((_BLK, D), lambda i: (i, 0)),
            pl.BlockSpec((_BLK, D), lambda i: (i, 0)),
        ],
        out_shape=[
            jax.ShapeDtypeStruct((N, D), jnp.float32),
            jax.ShapeDtypeStruct((N, D), jnp.float32),
        ],
    )(h, wl, wr, bl.reshape(1, D))


def _combine_body(act, aggA_ref, aggB_ref, cntA_ref, cntB_ref, z_ref, o_ref):
    cnt = jnp.maximum(cntA_ref[...] + cntB_ref[...], 1.0)
    agg = aggA_ref[0] + aggB_ref[0]
    h = agg / cnt + z_ref[...]
    if act == "relu":
        h = jnp.maximum(h, 0.0)
    elif act == "logsoftmax":
        m = jnp.max(h, axis=1, keepdims=True)
        h = h - m
        h = h - jnp.log(jnp.sum(jnp.exp(h), axis=1, keepdims=True))
    o_ref[...] = h


def _combine(agg2, cntA, cntB, z, act):
    grid = (N // _BLK,)
    return pl.pallas_call(
        functools.partial(_combine_body, act),
        grid=grid,
        in_specs=[
            pl.BlockSpec((1, _BLK, D), lambda i: (0, i, 0)),
            pl.BlockSpec((1, _BLK, D), lambda i: (1, i, 0)),
            pl.BlockSpec((_BLK, 1), lambda i: (i, 0)),
            pl.BlockSpec((_BLK, 1), lambda i: (i, 0)),
            pl.BlockSpec((_BLK, D), lambda i: (i, 0)),
        ],
        out_specs=pl.BlockSpec((_BLK, D), lambda i: (i, 0)),
        out_shape=jax.ShapeDtypeStruct((N, D), jnp.float32),
    )(agg2, agg2, cntA, cntB, z)


def _sc_body(with_counts, *refs):
    if with_counts:
        (ys_h, src_h, dst_h, zrows_h, zcnt_h,
         agg_h, cntA_h, cntB_h,
         acc_s, cntacc_s,
         src_v, dst_v, rows0_v, ones_v) = refs
    else:
        (ys_h, src_h, dst_h, zrows_h,
         agg_h,
         acc_s,
         src_v, dst_v, rows0_v) = refs

    cid = lax.axis_index("c")
    sid = lax.axis_index("s")
    wid = cid * NS + sid

    pltpu.sync_copy(zrows_h, acc_s.at[pl.ds(sid * ROWS_PT, ROWS_PT)])
    if with_counts:
        pltpu.sync_copy(zcnt_h, cntacc_s.at[pl.ds(sid * ROWS_PT, ROWS_PT)])

    pltpu.sync_copy(src_h.at[wid], src_v)
    pltpu.sync_copy(dst_h.at[wid], dst_v)

    if with_counts:
        for k in range(CH // 16):
            ones_v[pl.ds(k * 16, 16)] = jnp.full((16,), 1.0, jnp.float32)

    plsc.subcore_barrier()

    def chunk(j, _):
        pltpu.sync_copy(ys_h.at[src_v.at[j]], rows0_v)
        pltpu.sync_copy(rows0_v, acc_s.at[dst_v.at[j]], add=True)
        if with_counts:
            pltpu.sync_copy(ones_v, cntacc_s.at[dst_v.at[j]], add=True)
        return 0

    lax.fori_loop(0, NCHUNK, chunk, 0)

    plsc.subcore_barrier()

    pltpu.sync_copy(acc_s.at[pl.ds(sid * ROWS_PT, ROWS_PT)],
                    agg_h.at[cid].at[pl.ds(sid * ROWS_PT, ROWS_PT)])

    if with_counts:
        @pl.when(cid == 0)
        def _():
            pltpu.sync_copy(cntacc_s.at[pl.ds(sid * ROWS_PT, ROWS_PT)],
                            cntA_h.at[pl.ds(sid * ROWS_PT, ROWS_PT)])

        @pl.when(cid == 1)
        def _():
            pltpu.sync_copy(cntacc_s.at[pl.ds(sid * ROWS_PT, ROWS_PT)],
                            cntB_h.at[pl.ds(sid * ROWS_PT, ROWS_PT)])


def _sc_agg(ys, src2, dst2, zrows, zcnt):
    mesh = plsc.VectorSubcoreMesh(core_axis_name="c", subcore_axis_name="s")
    f = pl.kernel(
        functools.partial(_sc_body, True),
        out_type=[
            jax.ShapeDtypeStruct((2, NP, D), jnp.float32),
            jax.ShapeDtypeStruct((NP,), jnp.float32),
            jax.ShapeDtypeStruct((NP,), jnp.float32),
        ],
        mesh=mesh,
        scratch_types=[
            pltpu.VMEM_SHARED((NP, D), jnp.float32),
            pltpu.VMEM_SHARED((NP,), jnp.float32),
            pltpu.VMEM((NCHUNK, CH), jnp.int32),
            pltpu.VMEM((NCHUNK, CH), jnp.int32),
            pltpu.VMEM((CH, D), jnp.float32),
            pltpu.VMEM((CH,), jnp.float32),
        ],
    )
    return f(ys, src2, dst2, zrows, zcnt)


def _sc_agg_nocnt(ys, src2, dst2, zrows):
    mesh = plsc.VectorSubcoreMesh(core_axis_name="c", subcore_axis_name="s")
    f = pl.kernel(
        functools.partial(_sc_body, False),
        out_type=jax.ShapeDtypeStruct((2, NP, D), jnp.float32),
        mesh=mesh,
        scratch_types=[
            pltpu.VMEM_SHARED((NP, D), jnp.float32),
            pltpu.VMEM((NCHUNK, CH), jnp.int32),
            pltpu.VMEM((NCHUNK, CH), jnp.int32),
            pltpu.VMEM((CH, D), jnp.float32),
        ],
    )
    return f(ys, src2, dst2, zrows)


def _layer(h, wl, wr, bl, act, src2, dst2, zrows, zcnt, cnts):
    y, z = _mm(h, wl, wr, bl)
    if cnts is None:
        agg2, cntA, cntB = _sc_agg(y, src2, dst2, zrows, zcnt)
        cnts = (cntA.reshape(NP, 1), cntB.reshape(NP, 1))
    else:
        agg2 = _sc_agg_nocnt(y, src2, dst2, zrows)
    return _combine(agg2, cnts[0], cnts[1], z, act), cnts


def kernel(x, edge_index, W1l, W1r, W2l, W2r, W3l, W3r, b1l, b2l, b3l):
    npad = EPAD - E
    srcp = jnp.concatenate([edge_index[0], jnp.zeros((npad,), edge_index.dtype)])
    pad_dst = (N + (jnp.arange(npad, dtype=edge_index.dtype) % (NP - N)))
    dstp = jnp.concatenate([edge_index[1], pad_dst])
    src2 = srcp.reshape(NW, NCHUNK, CH)
    dst2 = dstp.reshape(NW, NCHUNK, CH)
    zrows = jnp.zeros((ROWS_PT, D), jnp.float32)
    zcnt = jnp.zeros((ROWS_PT,), jnp.float32)

    h, cnts = _layer(x, W1l, W1r, b1l, "relu", src2, dst2, zrows, zcnt, None)
    h, _ = _layer(h, W2l, W2r, b2l, "relu", src2, dst2, zrows, zcnt, cnts)
    out, _ = _layer(h, W3l, W3r, b3l, "logsoftmax", src2, dst2, zrows, zcnt, cnts)
    return out


# speedup vs baseline: 1.0518x; 1.0499x over previous
import functools

import jax
import jax.numpy as jnp
from jax import lax
from jax.experimental import pallas as pl
from jax.experimental.pallas import tpu as pltpu
from jax.experimental.pallas import tpu_sc as plsc

N = 10000
E = 320000
D = 128
NS = 16
NW = 2 * NS
CH = 80
NCHUNK = 128
EPAD = NW * NCHUNK * CH
NP = 10240
ROWS_PT = NP // NS

_BLK = 2000


def _mm_body(h_ref, wl_ref, wr_ref, bl_ref, y_ref, z_ref):
    h = h_ref[...]
    dn = (((1,), (1,)), ((), ()))
    y_ref[...] = lax.dot_general(h, wl_ref[...], dn,
                                 preferred_element_type=jnp.float32)
    z_ref[...] = lax.dot_general(h, wr_ref[...], dn,
                                 preferred_element_type=jnp.float32) + bl_ref[...]


def _mm(h, wl, wr, bl):
    grid = (N // _BLK,)
    return pl.pallas_call(
        _mm_body,
        grid=grid,
        in_specs=[
            pl.BlockSpec((_BLK, D), lambda i: (i, 0)),
            pl.BlockSpec((D, D), lambda i: (0, 0)),
            pl.BlockSpec((D, D), lambda i: (0, 0)),
            pl.BlockSpec((1, D), lambda i: (0, 0)),
        ],
        out_specs=[
            pl.BlockSpec((_BLK, D), lambda i: (i, 0)),
            pl.BlockSpec((_BLK, D), lambda i: (i, 0)),
        ],
        out_shape=[
            jax.ShapeDtypeStruct((N, D), jnp.float32),
            jax.ShapeDtypeStruct((N, D), jnp.float32),
        ],
    )(h, wl, wr, bl.reshape(1, D))


def _combine_body(act, aggA_ref, aggB_ref, cntA_ref, cntB_ref, z_ref, o_ref):
    cnt = jnp.maximum(cntA_ref[...] + cntB_ref[...], 1.0)
    agg = aggA_ref[0] + aggB_ref[0]
    h = agg / cnt + z_ref[...]
    if act == "relu":
        h = jnp.maximum(h, 0.0)
    elif act == "logsoftmax":
        m = jnp.max(h, axis=1, keepdims=True)
        h = h - m
        h = h - jnp.log(jnp.sum(jnp.exp(h), axis=1, keepdims=True))
    o_ref[...] = h


def _combine(agg2, cntA, cntB, z, act):
    grid = (N // _BLK,)
    return pl.pallas_call(
        functools.partial(_combine_body, act),
        grid=grid,
        in_specs=[
            pl.BlockSpec((1, _BLK, D), lambda i: (0, i, 0)),
            pl.BlockSpec((1, _BLK, D), lambda i: (1, i, 0)),
            pl.BlockSpec((_BLK, 1), lambda i: (i, 0)),
            pl.BlockSpec((_BLK, 1), lambda i: (i, 0)),
            pl.BlockSpec((_BLK, D), lambda i: (i, 0)),
        ],
        out_specs=pl.BlockSpec((_BLK, D), lambda i: (i, 0)),
        out_shape=jax.ShapeDtypeStruct((N, D), jnp.float32),
    )(agg2, agg2, cntA, cntB, z)


def _sc_body(with_counts, *refs):
    if with_counts:
        (ys_h, src_h, dst_h, zrows_h, zcnt_h,
         agg_h, cntA_h, cntB_h,
         acc_s, cntacc_s,
         src_v, dst_v, rows0_v, ones_v) = refs
    else:
        (ys_h, src_h, dst_h, zrows_h,
         agg_h,
         acc_s,
         src_v, dst_v, rows0_v, rows1_v, rsems) = refs

    cid = lax.axis_index("c")
    sid = lax.axis_index("s")
    wid = cid * NS + sid

    pltpu.sync_copy(zrows_h, acc_s.at[pl.ds(sid * ROWS_PT, ROWS_PT)])
    if with_counts:
        pltpu.sync_copy(zcnt_h, cntacc_s.at[pl.ds(sid * ROWS_PT, ROWS_PT)])

    pltpu.sync_copy(src_h.at[wid], src_v)
    pltpu.sync_copy(dst_h.at[wid], dst_v)

    if with_counts:
        for k in range(CH // 16):
            ones_v[pl.ds(k * 16, 16)] = jnp.full((16,), 1.0, jnp.float32)

    plsc.subcore_barrier()

    if with_counts:
        def chunk(j, _):
            pltpu.sync_copy(ys_h.at[src_v.at[pl.ds(j * CH, CH)]], rows0_v)
            pltpu.sync_copy(rows0_v, acc_s.at[dst_v.at[j]], add=True)
            pltpu.sync_copy(ones_v, cntacc_s.at[dst_v.at[j]], add=True)
            return 0

        lax.fori_loop(0, NCHUNK, chunk, 0)
    else:
        rows = (rows0_v, rows1_v)

        def g_start(c, b):
            pltpu.async_copy(ys_h.at[src_v.at[pl.ds(c * CH, CH)]], rows[b],
                            rsems.at[b])

        def g_wait(b):
            pltpu.make_async_copy(ys_h.at[src_v.at[pl.ds(0, CH)]], rows[b],
                                  rsems.at[b]).wait()

        def scat(c, b):
            pltpu.sync_copy(rows[b], acc_s.at[dst_v.at[c]], add=True)

        g_start(0, 0)

        def pair(i, _):
            c0 = 2 * i
            g_start(c0 + 1, 1)
            g_wait(0)
            scat(c0, 0)

            @pl.when(c0 + 2 < NCHUNK)
            def _():
                g_start(c0 + 2, 0)

            g_wait(1)
            scat(c0 + 1, 1)
            return 0

        lax.fori_loop(0, NCHUNK // 2, pair, 0)

    plsc.subcore_barrier()

    pltpu.sync_copy(acc_s.at[pl.ds(sid * ROWS_PT, ROWS_PT)],
                    agg_h.at[cid].at[pl.ds(sid * ROWS_PT, ROWS_PT)])

    if with_counts:
        @pl.when(cid == 0)
        def _():
            pltpu.sync_copy(cntacc_s.at[pl.ds(sid * ROWS_PT, ROWS_PT)],
                            cntA_h.at[pl.ds(sid * ROWS_PT, ROWS_PT)])

        @pl.when(cid == 1)
        def _():
            pltpu.sync_copy(cntacc_s.at[pl.ds(sid * ROWS_PT, ROWS_PT)],
                            cntB_h.at[pl.ds(sid * ROWS_PT, ROWS_PT)])


def _sc_agg(ys, src2, dst2, zrows, zcnt):
    mesh = plsc.VectorSubcoreMesh(core_axis_name="c", subcore_axis_name="s")
    f = pl.kernel(
        functools.partial(_sc_body, True),
        out_type=[
            jax.ShapeDtypeStruct((2, NP, D), jnp.float32),
            jax.ShapeDtypeStruct((NP,), jnp.float32),
            jax.ShapeDtypeStruct((NP,), jnp.float32),
        ],
        mesh=mesh,
        scratch_types=[
            pltpu.VMEM_SHARED((NP, D), jnp.float32),
            pltpu.VMEM_SHARED((NP,), jnp.float32),
            pltpu.VMEM((NCHUNK * CH,), jnp.int32),
            pltpu.VMEM((NCHUNK, CH), jnp.int32),
            pltpu.VMEM((CH, D), jnp.float32),
            pltpu.VMEM((CH,), jnp.float32),
        ],
    )
    return f(ys, src2, dst2, zrows, zcnt)


def _sc_agg_nocnt(ys, src2, dst2, zrows):
    mesh = plsc.VectorSubcoreMesh(core_axis_name="c", subcore_axis_name="s")
    f = pl.kernel(
        functools.partial(_sc_body, False),
        out_type=jax.ShapeDtypeStruct((2, NP, D), jnp.float32),
        mesh=mesh,
        scratch_types=[
            pltpu.VMEM_SHARED((NP, D), jnp.float32),
            pltpu.VMEM((NCHUNK * CH,), jnp.int32),
            pltpu.VMEM((NCHUNK, CH), jnp.int32),
            pltpu.VMEM((CH, D), jnp.float32),
            pltpu.VMEM((CH, D), jnp.float32),
            pltpu.SemaphoreType.DMA((2,)),
        ],
    )
    return f(ys, src2, dst2, zrows)


def _layer(h, wl, wr, bl, act, src2, dst2, zrows, zcnt, cnts):
    y, z = _mm(h, wl, wr, bl)
    if cnts is None:
        agg2, cntA, cntB = _sc_agg(y, src2, dst2, zrows, zcnt)
        cnts = (cntA.reshape(NP, 1), cntB.reshape(NP, 1))
    else:
        agg2 = _sc_agg_nocnt(y, src2, dst2, zrows)
    return _combine(agg2, cnts[0], cnts[1], z, act), cnts


def kernel(x, edge_index, W1l, W1r, W2l, W2r, W3l, W3r, b1l, b2l, b3l):
    npad = EPAD - E
    srcp = jnp.concatenate([edge_index[0], jnp.zeros((npad,), edge_index.dtype)])
    pad_dst = (N + (jnp.arange(npad, dtype=edge_index.dtype) % (NP - N)))
    dstp = jnp.concatenate([edge_index[1], pad_dst])
    src2 = srcp.reshape(NW, NCHUNK * CH)
    dst2 = dstp.reshape(NW, NCHUNK, CH)
    zrows = jnp.zeros((ROWS_PT, D), jnp.float32)
    zcnt = jnp.zeros((ROWS_PT,), jnp.float32)

    h, cnts = _layer(x, W1l, W1r, b1l, "relu", src2, dst2, zrows, zcnt, None)
    h, _ = _layer(h, W2l, W2r, b2l, "relu", src2, dst2, zrows, zcnt, cnts)
    out, _ = _layer(h, W3l, W3r, b3l, "logsoftmax", src2, dst2, zrows, zcnt, cnts)
    return out


# speedup vs baseline: 2.9081x; 2.7649x over previous
import functools

import jax
import jax.numpy as jnp
from jax import lax
from jax.experimental import pallas as pl
from jax.experimental.pallas import tpu as pltpu
from jax.experimental.pallas import tpu_sc as plsc

N = 10000
E = 320000
D = 128
NS = 16
NW = 2 * NS
CH = 80
NCHUNK = 125
NP = 10240
ROWS_PT = NP // NS

_BLK = 2000


def _mm_body(h_ref, wl_ref, wr_ref, bl_ref, y_ref, z_ref):
    h = h_ref[...]
    dn = (((1,), (1,)), ((), ()))
    y_ref[...] = lax.dot_general(h, wl_ref[...], dn,
                                 preferred_element_type=jnp.float32)
    z_ref[...] = lax.dot_general(h, wr_ref[...], dn,
                                 preferred_element_type=jnp.float32) + bl_ref[...]


def _mm(h, wl, wr, bl):
    grid = (N // _BLK,)
    return pl.pallas_call(
        _mm_body,
        grid=grid,
        in_specs=[
            pl.BlockSpec((_BLK, D), lambda i: (i, 0)),
            pl.BlockSpec((D, D), lambda i: (0, 0)),
            pl.BlockSpec((D, D), lambda i: (0, 0)),
            pl.BlockSpec((1, D), lambda i: (0, 0)),
        ],
        out_specs=[
            pl.BlockSpec((_BLK, D), lambda i: (i, 0)),
            pl.BlockSpec((_BLK, D), lambda i: (i, 0)),
        ],
        out_shape=[
            jax.ShapeDtypeStruct((N, D), jnp.float32),
            jax.ShapeDtypeStruct((N, D), jnp.float32),
        ],
    )(h, wl, wr, bl.reshape(1, D))


def _combine_body(act, aggA_ref, aggB_ref, cntA_ref, cntB_ref, z_ref, o_ref):
    cnt = jnp.maximum(cntA_ref[...] + cntB_ref[...], 1.0)
    agg = aggA_ref[0] + aggB_ref[0]
    h = agg / cnt + z_ref[...]
    if act == "relu":
        h = jnp.maximum(h, 0.0)
    elif act == "logsoftmax":
        m = jnp.max(h, axis=1, keepdims=True)
        h = h - m
        h = h - jnp.log(jnp.sum(jnp.exp(h), axis=1, keepdims=True))
    o_ref[...] = h


def _combine(agg2, cntA, cntB, z, act):
    grid = (N // _BLK,)
    return pl.pallas_call(
        functools.partial(_combine_body, act),
        grid=grid,
        in_specs=[
            pl.BlockSpec((1, _BLK, D), lambda i: (0, i, 0)),
            pl.BlockSpec((1, _BLK, D), lambda i: (1, i, 0)),
            pl.BlockSpec((_BLK, 1), lambda i: (i, 0)),
            pl.BlockSpec((_BLK, 1), lambda i: (i, 0)),
            pl.BlockSpec((_BLK, D), lambda i: (i, 0)),
        ],
        out_specs=pl.BlockSpec((_BLK, D), lambda i: (i, 0)),
        out_shape=jax.ShapeDtypeStruct((N, D), jnp.float32),
    )(agg2, agg2, cntA, cntB, z)


def _sc_body(with_counts, *refs):
    if with_counts:
        (ys_h, src_h, dst_h, zrows_h, zcnt_h,
         agg_h, cntA_h, cntB_h,
         acc_s, cntacc_s,
         src_v, dst_v, rows0_v, ones_v) = refs
    else:
        (ys_h, src_h, dst_h, zrows_h,
         agg_h,
         acc_s,
         src_v, dst_v, rows0_v, rows1_v, rsems) = refs

    cid = lax.axis_index("c")
    sid = lax.axis_index("s")
    wid = cid * NS + sid

    pltpu.sync_copy(zrows_h, acc_s.at[pl.ds(sid * ROWS_PT, ROWS_PT)])
    if with_counts:
        pltpu.sync_copy(zcnt_h, cntacc_s.at[pl.ds(sid * ROWS_PT, ROWS_PT)])

    pltpu.sync_copy(src_h.at[wid], src_v)
    pltpu.sync_copy(dst_h.at[wid], dst_v)

    if with_counts:
        for k in range(CH // 16):
            ones_v[pl.ds(k * 16, 16)] = jnp.full((16,), 1.0, jnp.float32)

    plsc.subcore_barrier()

    if with_counts:
        def chunk(j, _):
            pltpu.sync_copy(ys_h.at[src_v.at[pl.ds(j * CH, CH)]], rows0_v)
            pltpu.sync_copy(rows0_v, acc_s.at[dst_v.at[j]], add=True)
            pltpu.sync_copy(ones_v, cntacc_s.at[dst_v.at[j]], add=True)
            return 0

        lax.fori_loop(0, NCHUNK, chunk, 0)
    else:
        rows = (rows0_v, rows1_v)

        def g_start(c, b):
            pltpu.async_copy(ys_h.at[src_v.at[pl.ds(c * CH, CH)]], rows[b],
                            rsems.at[b])

        def g_wait(b):
            pltpu.make_async_copy(ys_h.at[src_v.at[pl.ds(0, CH)]], rows[b],
                                  rsems.at[b]).wait()

        def scat(c, b):
            pltpu.sync_copy(rows[b], acc_s.at[dst_v.at[c]], add=True)

        g_start(0, 0)

        def pair(i, _):
            c0 = 2 * i
            g_start(c0 + 1, 1)
            g_wait(0)
            scat(c0, 0)

            @pl.when(c0 + 2 < NCHUNK)
            def _():
                g_start(c0 + 2, 0)

            g_wait(1)
            scat(c0 + 1, 1)
            return 0

        lax.fori_loop(0, NCHUNK // 2, pair, 0)
        g_wait(0)
        scat(NCHUNK - 1, 0)

    plsc.subcore_barrier()

    pltpu.sync_copy(acc_s.at[pl.ds(sid * ROWS_PT, ROWS_PT)],
                    agg_h.at[cid].at[pl.ds(sid * ROWS_PT, ROWS_PT)])

    if with_counts:
        @pl.when(cid == 0)
        def _():
            pltpu.sync_copy(cntacc_s.at[pl.ds(sid * ROWS_PT, ROWS_PT)],
                            cntA_h.at[pl.ds(sid * ROWS_PT, ROWS_PT)])

        @pl.when(cid == 1)
        def _():
            pltpu.sync_copy(cntacc_s.at[pl.ds(sid * ROWS_PT, ROWS_PT)],
                            cntB_h.at[pl.ds(sid * ROWS_PT, ROWS_PT)])


def _sc_agg(ys, src2, dst2, zrows, zcnt):
    mesh = plsc.VectorSubcoreMesh(core_axis_name="c", subcore_axis_name="s")
    f = pl.kernel(
        functools.partial(_sc_body, True),
        out_type=[
            jax.ShapeDtypeStruct((2, NP, D), jnp.float32),
            jax.ShapeDtypeStruct((NP,), jnp.float32),
            jax.ShapeDtypeStruct((NP,), jnp.float32),
        ],
        mesh=mesh,
        scratch_types=[
            pltpu.VMEM_SHARED((NP, D), jnp.float32),
            pltpu.VMEM_SHARED((NP,), jnp.float32),
            pltpu.VMEM((NCHUNK * CH,), jnp.int32),
            pltpu.VMEM((NCHUNK, CH), jnp.int32),
            pltpu.VMEM((CH, D), jnp.float32),
            pltpu.VMEM((CH,), jnp.float32),
        ],
    )
    return f(ys, src2, dst2, zrows, zcnt)


def _sc_agg_nocnt(ys, src2, dst2, zrows):
    mesh = plsc.VectorSubcoreMesh(core_axis_name="c", subcore_axis_name="s")
    f = pl.kernel(
        functools.partial(_sc_body, False),
        out_type=jax.ShapeDtypeStruct((2, NP, D), jnp.float32),
        mesh=mesh,
        scratch_types=[
            pltpu.VMEM_SHARED((NP, D), jnp.float32),
            pltpu.VMEM((NCHUNK * CH,), jnp.int32),
            pltpu.VMEM((NCHUNK, CH), jnp.int32),
            pltpu.VMEM((CH, D), jnp.float32),
            pltpu.VMEM((CH, D), jnp.float32),
            pltpu.SemaphoreType.DMA((2,)),
        ],
    )
    return f(ys, src2, dst2, zrows)


def _layer(h, wl, wr, bl, act, src2, dst2, zrows, zcnt, cnts):
    y, z = _mm(h, wl, wr, bl)
    if cnts is None:
        agg2, cntA, cntB = _sc_agg(y, src2, dst2, zrows, zcnt)
        cnts = (cntA.reshape(NP, 1), cntB.reshape(NP, 1))
    else:
        agg2 = _sc_agg_nocnt(y, src2, dst2, zrows)
    return _combine(agg2, cnts[0], cnts[1], z, act), cnts


def kernel(x, edge_index, W1l, W1r, W2l, W2r, W3l, W3r, b1l, b2l, b3l):
    src2 = edge_index[0].reshape(NW, NCHUNK * CH)
    dst2 = edge_index[1].reshape(NW, NCHUNK, CH)
    zrows = jnp.zeros((ROWS_PT, D), jnp.float32)
    zcnt = jnp.zeros((ROWS_PT,), jnp.float32)

    h, cnts = _layer(x, W1l, W1r, b1l, "relu", src2, dst2, zrows, zcnt, None)
    h, _ = _layer(h, W2l, W2r, b2l, "relu", src2, dst2, zrows, zcnt, cnts)
    out, _ = _layer(h, W3l, W3r, b3l, "logsoftmax", src2, dst2, zrows, zcnt, cnts)
    return out


# speedup vs baseline: 3.4917x; 1.2007x over previous
import functools

import jax
import jax.numpy as jnp
from jax import lax
from jax.experimental import pallas as pl
from jax.experimental.pallas import tpu as pltpu
from jax.experimental.pallas import tpu_sc as plsc

N = 10000
E = 320000
D = 128
NS = 16
NW = 2 * NS
CH = 80
NCHUNK = 125
NP = 10240
ROWS_PT = NP // NS

_BLK = 2000


def _mm_body(h_ref, wl_ref, wr_ref, bl_ref, y_ref, z_ref):
    h = h_ref[...]
    dn = (((1,), (1,)), ((), ()))
    y_ref[...] = lax.dot_general(h, wl_ref[...], dn,
                                 preferred_element_type=jnp.float32)
    z_ref[...] = lax.dot_general(h, wr_ref[...], dn,
                                 preferred_element_type=jnp.float32) + bl_ref[...]


def _mm(h, wl, wr, bl):
    grid = (N // _BLK,)
    return pl.pallas_call(
        _mm_body,
        grid=grid,
        in_specs=[
            pl.BlockSpec((_BLK, D), lambda i: (i, 0)),
            pl.BlockSpec((D, D), lambda i: (0, 0)),
            pl.BlockSpec((D, D), lambda i: (0, 0)),
            pl.BlockSpec((1, D), lambda i: (0, 0)),
        ],
        out_specs=[
            pl.BlockSpec((_BLK, D), lambda i: (i, 0)),
            pl.BlockSpec((_BLK, D), lambda i: (i, 0)),
        ],
        out_shape=[
            jax.ShapeDtypeStruct((N, D), jnp.float32),
            jax.ShapeDtypeStruct((N, D), jnp.float32),
        ],
    )(h, wl, wr, bl.reshape(1, D))


def _combine_body(act, aggA_ref, aggB_ref, cntA_ref, cntB_ref, z_ref, o_ref):
    cnt = jnp.maximum(cntA_ref[...] + cntB_ref[...], 1.0)
    agg = aggA_ref[0] + aggB_ref[0]
    h = agg / cnt + z_ref[...]
    if act == "relu":
        h = jnp.maximum(h, 0.0)
    elif act == "logsoftmax":
        m = jnp.max(h, axis=1, keepdims=True)
        h = h - m
        h = h - jnp.log(jnp.sum(jnp.exp(h), axis=1, keepdims=True))
    o_ref[...] = h


def _combine(agg2, cntA, cntB, z, act):
    grid = (N // _BLK,)
    return pl.pallas_call(
        functools.partial(_combine_body, act),
        grid=grid,
        in_specs=[
            pl.BlockSpec((1, _BLK, D), lambda i: (0, i, 0)),
            pl.BlockSpec((1, _BLK, D), lambda i: (1, i, 0)),
            pl.BlockSpec((_BLK, 1), lambda i: (i, 0)),
            pl.BlockSpec((_BLK, 1), lambda i: (i, 0)),
            pl.BlockSpec((_BLK, D), lambda i: (i, 0)),
        ],
        out_specs=pl.BlockSpec((_BLK, D), lambda i: (i, 0)),
        out_shape=jax.ShapeDtypeStruct((N, D), jnp.float32),
    )(agg2, agg2, cntA, cntB, z)


def _sc_body(with_counts, *refs):
    if with_counts:
        (ys_h, src_h, dst_h, zrows_h, zcnt_h,
         agg_h, cntA_h, cntB_h,
         acc_s, cntacc_s,
         src_v, dst_v, rows0_v, rows1_v, ones_v, rsems) = refs
    else:
        (ys_h, src_h, dst_h, zrows_h,
         agg_h,
         acc_s,
         src_v, dst_v, rows0_v, rows1_v, rsems) = refs

    cid = lax.axis_index("c")
    sid = lax.axis_index("s")
    wid = cid * NS + sid

    pltpu.sync_copy(zrows_h, acc_s.at[pl.ds(sid * ROWS_PT, ROWS_PT)])
    if with_counts:
        pltpu.sync_copy(zcnt_h, cntacc_s.at[pl.ds(sid * ROWS_PT, ROWS_PT)])

    pltpu.sync_copy(src_h.at[wid], src_v)
    pltpu.sync_copy(dst_h.at[wid], dst_v)

    if with_counts:
        for k in range(CH // 16):
            ones_v[pl.ds(k * 16, 16)] = jnp.full((16,), 1.0, jnp.float32)

    plsc.subcore_barrier()

    rows = (rows0_v, rows1_v)

    def g_start(c, b):
        pltpu.async_copy(ys_h.at[src_v.at[pl.ds(c * CH, CH)]], rows[b],
                        rsems.at[b])

    def g_wait(b):
        pltpu.make_async_copy(ys_h.at[src_v.at[pl.ds(0, CH)]], rows[b],
                              rsems.at[b]).wait()

    def scat(c, b):
        pltpu.sync_copy(rows[b], acc_s.at[dst_v.at[c]], add=True)
        if with_counts:
            pltpu.sync_copy(ones_v, cntacc_s.at[dst_v.at[c]], add=True)

    g_start(0, 0)

    def pair(i, _):
        c0 = 2 * i
        g_start(c0 + 1, 1)
        g_wait(0)
        scat(c0, 0)

        @pl.when(c0 + 2 < NCHUNK)
        def _():
            g_start(c0 + 2, 0)

        g_wait(1)
        scat(c0 + 1, 1)
        return 0

    lax.fori_loop(0, NCHUNK // 2, pair, 0)
    g_wait(0)
    scat(NCHUNK - 1, 0)

    plsc.subcore_barrier()

    pltpu.sync_copy(acc_s.at[pl.ds(sid * ROWS_PT, ROWS_PT)],
                    agg_h.at[cid].at[pl.ds(sid * ROWS_PT, ROWS_PT)])

    if with_counts:
        @pl.when(cid == 0)
        def _():
            pltpu.sync_copy(cntacc_s.at[pl.ds(sid * ROWS_PT, ROWS_PT)],
                            cntA_h.at[pl.ds(sid * ROWS_PT, ROWS_PT)])

        @pl.when(cid == 1)
        def _():
            pltpu.sync_copy(cntacc_s.at[pl.ds(sid * ROWS_PT, ROWS_PT)],
                            cntB_h.at[pl.ds(sid * ROWS_PT, ROWS_PT)])


def _sc_agg(ys, src2, dst2, zrows, zcnt):
    mesh = plsc.VectorSubcoreMesh(core_axis_name="c", subcore_axis_name="s")
    f = pl.kernel(
        functools.partial(_sc_body, True),
        out_type=[
            jax.ShapeDtypeStruct((2, NP, D), jnp.float32),
            jax.ShapeDtypeStruct((NP,), jnp.float32),
            jax.ShapeDtypeStruct((NP,), jnp.float32),
        ],
        mesh=mesh,
        scratch_types=[
            pltpu.VMEM_SHARED((NP, D), jnp.float32),
            pltpu.VMEM_SHARED((NP,), jnp.float32),
            pltpu.VMEM((NCHUNK * CH,), jnp.int32),
            pltpu.VMEM((NCHUNK, CH), jnp.int32),
            pltpu.VMEM((CH, D), jnp.float32),
            pltpu.VMEM((CH, D), jnp.float32),
            pltpu.VMEM((CH,), jnp.float32),
            pltpu.SemaphoreType.DMA((2,)),
        ],
    )
    return f(ys, src2, dst2, zrows, zcnt)


def _sc_agg_nocnt(ys, src2, dst2, zrows):
    mesh = plsc.VectorSubcoreMesh(core_axis_name="c", subcore_axis_name="s")
    f = pl.kernel(
        functools.partial(_sc_body, False),
        out_type=jax.ShapeDtypeStruct((2, NP, D), jnp.float32),
        mesh=mesh,
        scratch_types=[
            pltpu.VMEM_SHARED((NP, D), jnp.float32),
            pltpu.VMEM((NCHUNK * CH,), jnp.int32),
            pltpu.VMEM((NCHUNK, CH), jnp.int32),
            pltpu.VMEM((CH, D), jnp.float32),
            pltpu.VMEM((CH, D), jnp.float32),
            pltpu.SemaphoreType.DMA((2,)),
        ],
    )
    return f(ys, src2, dst2, zrows)


def _layer(h, wl, wr, bl, act, src2, dst2, zrows, zcnt, cnts):
    y, z = _mm(h, wl, wr, bl)
    if cnts is None:
        agg2, cntA, cntB = _sc_agg(y, src2, dst2, zrows, zcnt)
        cnts = (cntA.reshape(NP, 1), cntB.reshape(NP, 1))
    else:
        agg2 = _sc_agg_nocnt(y, src2, dst2, zrows)
    return _combine(agg2, cnts[0], cnts[1], z, act), cnts


def kernel(x, edge_index, W1l, W1r, W2l, W2r, W3l, W3r, b1l, b2l, b3l):
    src2 = edge_index[0].reshape(NW, NCHUNK * CH)
    dst2 = edge_index[1].reshape(NW, NCHUNK, CH)
    zrows = jnp.zeros((ROWS_PT, D), jnp.float32)
    zcnt = jnp.zeros((ROWS_PT,), jnp.float32)

    h, cnts = _layer(x, W1l, W1r, b1l, "relu", src2, dst2, zrows, zcnt, None)
    h, _ = _layer(h, W2l, W2r, b2l, "relu", src2, dst2, zrows, zcnt, cnts)
    out, _ = _layer(h, W3l, W3r, b3l, "logsoftmax", src2, dst2, zrows, zcnt, cnts)
    return out


# speedup vs baseline: 3.6054x; 1.0326x over previous
import functools

import jax
import jax.numpy as jnp
from jax import lax
from jax.experimental import pallas as pl
from jax.experimental.pallas import tpu as pltpu
from jax.experimental.pallas import tpu_sc as plsc

N = 10000
E = 320000
D = 128
NS = 16
NW = 2 * NS
CH = 80
NCHUNK = 125
NP = 10240
ROWS_PT = NP // NS

_BLK = 2000


def _mm_body(h_ref, wl_ref, wr_ref, bl_ref, y_ref, z_ref):
    h = h_ref[...]
    dn = (((1,), (1,)), ((), ()))
    y_ref[...] = lax.dot_general(h, wl_ref[...], dn,
                                 preferred_element_type=jnp.float32)
    z_ref[...] = lax.dot_general(h, wr_ref[...], dn,
                                 preferred_element_type=jnp.float32) + bl_ref[...]


def _mm(h, wl, wr, bl):
    grid = (N // _BLK,)
    return pl.pallas_call(
        _mm_body,
        grid=grid,
        in_specs=[
            pl.BlockSpec((_BLK, D), lambda i: (i, 0)),
            pl.BlockSpec((D, D), lambda i: (0, 0)),
            pl.BlockSpec((D, D), lambda i: (0, 0)),
            pl.BlockSpec((1, D), lambda i: (0, 0)),
        ],
        out_specs=[
            pl.BlockSpec((_BLK, D), lambda i: (i, 0)),
            pl.BlockSpec((_BLK, D), lambda i: (i, 0)),
        ],
        out_shape=[
            jax.ShapeDtypeStruct((N, D), jnp.float32),
            jax.ShapeDtypeStruct((N, D), jnp.float32),
        ],
    )(h, wl, wr, bl.reshape(1, D))


def _combine_body(act, aggA_ref, aggB_ref, cntA_ref, cntB_ref, z_ref, o_ref):
    cnt = jnp.maximum(cntA_ref[...] + cntB_ref[...], 1.0)
    agg = aggA_ref[0] + aggB_ref[0]
    h = agg / cnt + z_ref[...]
    if act == "relu":
        h = jnp.maximum(h, 0.0)
    elif act == "logsoftmax":
        m = jnp.max(h, axis=1, keepdims=True)
        h = h - m
        h = h - jnp.log(jnp.sum(jnp.exp(h), axis=1, keepdims=True))
    o_ref[...] = h


def _combine(agg2, cntA, cntB, z, act):
    grid = (N // _BLK,)
    return pl.pallas_call(
        functools.partial(_combine_body, act),
        grid=grid,
        in_specs=[
            pl.BlockSpec((1, _BLK, D), lambda i: (0, i, 0)),
            pl.BlockSpec((1, _BLK, D), lambda i: (1, i, 0)),
            pl.BlockSpec((_BLK, 1), lambda i: (i, 0)),
            pl.BlockSpec((_BLK, 1), lambda i: (i, 0)),
            pl.BlockSpec((_BLK, D), lambda i: (i, 0)),
        ],
        out_specs=pl.BlockSpec((_BLK, D), lambda i: (i, 0)),
        out_shape=jax.ShapeDtypeStruct((N, D), jnp.float32),
    )(agg2, agg2, cntA, cntB, z)


def _fused_body(aggA_ref, aggB_ref, cntA_ref, cntB_ref, z_ref,
                wl_ref, wr_ref, bl_ref, y_ref, zo_ref):
    cnt = jnp.maximum(cntA_ref[...] + cntB_ref[...], 1.0)
    h = jnp.maximum((aggA_ref[0] + aggB_ref[0]) / cnt + z_ref[...], 0.0)
    dn = (((1,), (1,)), ((), ()))
    y_ref[...] = lax.dot_general(h, wl_ref[...], dn,
                                 preferred_element_type=jnp.float32)
    zo_ref[...] = lax.dot_general(h, wr_ref[...], dn,
                                  preferred_element_type=jnp.float32) + bl_ref[...]


def _fused(agg2, cntA, cntB, z, wl, wr, bl):
    grid = (N // _BLK,)
    return pl.pallas_call(
        _fused_body,
        grid=grid,
        in_specs=[
            pl.BlockSpec((1, _BLK, D), lambda i: (0, i, 0)),
            pl.BlockSpec((1, _BLK, D), lambda i: (1, i, 0)),
            pl.BlockSpec((_BLK, 1), lambda i: (i, 0)),
            pl.BlockSpec((_BLK, 1), lambda i: (i, 0)),
            pl.BlockSpec((_BLK, D), lambda i: (i, 0)),
            pl.BlockSpec((D, D), lambda i: (0, 0)),
            pl.BlockSpec((D, D), lambda i: (0, 0)),
            pl.BlockSpec((1, D), lambda i: (0, 0)),
        ],
        out_specs=[
            pl.BlockSpec((_BLK, D), lambda i: (i, 0)),
            pl.BlockSpec((_BLK, D), lambda i: (i, 0)),
        ],
        out_shape=[
            jax.ShapeDtypeStruct((N, D), jnp.float32),
            jax.ShapeDtypeStruct((N, D), jnp.float32),
        ],
    )(agg2, agg2, cntA, cntB, z, wl, wr, bl.reshape(1, D))


def _sc_body(with_counts, *refs):
    if with_counts:
        (ys_h, src_h, dst_h, zrows_h, zcnt_h,
         agg_h, cntA_h, cntB_h,
         acc_s, cntacc_s,
         src_v, dst_v, rows0_v, rows1_v, ones_v, rsems) = refs
    else:
        (ys_h, src_h, dst_h, zrows_h,
         agg_h,
         acc_s,
         src_v, dst_v, rows0_v, rows1_v, rsems) = refs

    cid = lax.axis_index("c")
    sid = lax.axis_index("s")
    wid = cid * NS + sid

    pltpu.sync_copy(zrows_h, acc_s.at[pl.ds(sid * ROWS_PT, ROWS_PT)])
    if with_counts:
        pltpu.sync_copy(zcnt_h, cntacc_s.at[pl.ds(sid * ROWS_PT, ROWS_PT)])

    pltpu.sync_copy(src_h.at[wid], src_v)
    pltpu.sync_copy(dst_h.at[wid], dst_v)

    if with_counts:
        for k in range(CH // 16):
            ones_v[pl.ds(k * 16, 16)] = jnp.full((16,), 1.0, jnp.float32)

    plsc.subcore_barrier()

    rows = (rows0_v, rows1_v)

    def g_start(c, b):
        pltpu.async_copy(ys_h.at[src_v.at[pl.ds(c * CH, CH)]], rows[b],
                        rsems.at[b])

    def g_wait(b):
        pltpu.make_async_copy(ys_h.at[src_v.at[pl.ds(0, CH)]], rows[b],
                              rsems.at[b]).wait()

    def scat(c, b):
        pltpu.sync_copy(rows[b], acc_s.at[dst_v.at[c]], add=True)
        if with_counts:
            pltpu.sync_copy(ones_v, cntacc_s.at[dst_v.at[c]], add=True)

    g_start(0, 0)

    def pair(i, _):
        c0 = 2 * i
        g_start(c0 + 1, 1)
        g_wait(0)
        scat(c0, 0)

        @pl.when(c0 + 2 < NCHUNK)
        def _():
            g_start(c0 + 2, 0)

        g_wait(1)
        scat(c0 + 1, 1)
        return 0

    lax.fori_loop(0, NCHUNK // 2, pair, 0)
    g_wait(0)
    scat(NCHUNK - 1, 0)

    plsc.subcore_barrier()

    pltpu.sync_copy(acc_s.at[pl.ds(sid * ROWS_PT, ROWS_PT)],
                    agg_h.at[cid].at[pl.ds(sid * ROWS_PT, ROWS_PT)])

    if with_counts:
        @pl.when(cid == 0)
        def _():
            pltpu.sync_copy(cntacc_s.at[pl.ds(sid * ROWS_PT, ROWS_PT)],
                            cntA_h.at[pl.ds(sid * ROWS_PT, ROWS_PT)])

        @pl.when(cid == 1)
        def _():
            pltpu.sync_copy(cntacc_s.at[pl.ds(sid * ROWS_PT, ROWS_PT)],
                            cntB_h.at[pl.ds(sid * ROWS_PT, ROWS_PT)])


def _sc_agg(ys, src2, dst2, zrows, zcnt):
    mesh = plsc.VectorSubcoreMesh(core_axis_name="c", subcore_axis_name="s")
    f = pl.kernel(
        functools.partial(_sc_body, True),
        out_type=[
            jax.ShapeDtypeStruct((2, NP, D), jnp.float32),
            jax.ShapeDtypeStruct((NP,), jnp.float32),
            jax.ShapeDtypeStruct((NP,), jnp.float32),
        ],
        mesh=mesh,
        scratch_types=[
            pltpu.VMEM_SHARED((NP, D), jnp.float32),
            pltpu.VMEM_SHARED((NP,), jnp.float32),
            pltpu.VMEM((NCHUNK * CH,), jnp.int32),
            pltpu.VMEM((NCHUNK, CH), jnp.int32),
            pltpu.VMEM((CH, D), jnp.float32),
            pltpu.VMEM((CH, D), jnp.float32),
            pltpu.VMEM((CH,), jnp.float32),
            pltpu.SemaphoreType.DMA((2,)),
        ],
    )
    return f(ys, src2, dst2, zrows, zcnt)


def _sc_agg_nocnt(ys, src2, dst2, zrows):
    mesh = plsc.VectorSubcoreMesh(core_axis_name="c", subcore_axis_name="s")
    f = pl.kernel(
        functools.partial(_sc_body, False),
        out_type=jax.ShapeDtypeStruct((2, NP, D), jnp.float32),
        mesh=mesh,
        scratch_types=[
            pltpu.VMEM_SHARED((NP, D), jnp.float32),
            pltpu.VMEM((NCHUNK * CH,), jnp.int32),
            pltpu.VMEM((NCHUNK, CH), jnp.int32),
            pltpu.VMEM((CH, D), jnp.float32),
            pltpu.VMEM((CH, D), jnp.float32),
            pltpu.SemaphoreType.DMA((2,)),
        ],
    )
    return f(ys, src2, dst2, zrows)


def kernel(x, edge_index, W1l, W1r, W2l, W2r, W3l, W3r, b1l, b2l, b3l):
    src2 = edge_index[0].reshape(NW, NCHUNK * CH)
    dst2 = edge_index[1].reshape(NW, NCHUNK, CH)
    zrows = jnp.zeros((ROWS_PT, D), jnp.float32)
    zcnt = jnp.zeros((ROWS_PT,), jnp.float32)

    y1, z1 = _mm(x, W1l, W1r, b1l)
    agg1, cntA, cntB = _sc_agg(y1, src2, dst2, zrows, zcnt)
    cA, cB = cntA.reshape(NP, 1), cntB.reshape(NP, 1)
    y2, z2 = _fused(agg1, cA, cB, z1, W2l, W2r, b2l)
    agg2 = _sc_agg_nocnt(y2, src2, dst2, zrows)
    y3, z3 = _fused(agg2, cA, cB, z2, W3l, W3r, b3l)
    agg3 = _sc_agg_nocnt(y3, src2, dst2, zrows)
    return _combine(agg3, cA, cB, z3, "logsoftmax")
